# baseline (speedup 1.0000x reference)
import functools

import jax
import jax.numpy as jnp
from jax.experimental import pallas as pl
from jax.experimental.pallas import tpu as pltpu

N = 10000
D = 128
TM = 200

_EPS = 1e-5


def _artanh(x):
    x = jnp.clip(x, -1.0 + 1e-7, 1.0 - 1e-7)
    return 0.5 * (jnp.log1p(x) - jnp.log1p(-x))


def _rownorm(x):
    return jnp.maximum(
        jnp.sqrt(jnp.sum(x * x, axis=-1, keepdims=True)), 1e-15
    )


def _proj(x):
    n = _rownorm(x)
    maxn = 1.0 - _EPS
    return jnp.where(n > maxn, x / n * maxn, x)


def _prologue_kernel(x_ref, w_ref, he_ref):
    x = x_ref[...]
    w = w_ref[...]
    xn = _rownorm(x)
    mx = jax.lax.dot_general(
        x, w, (((1,), (1,)), ((), ())), preferred_element_type=jnp.float32
    )
    mxn = _rownorm(mx)
    hidden = jnp.tanh(mxn / xn * _artanh(xn)) * mx / mxn
    pn = _rownorm(hidden)
    he_ref[...] = _artanh(pn) * hidden / pn


def _main_kernel(adj_ref, he_ref, out_ref):
    s = jnp.dot(
        adj_ref[...], he_ref[...], preferred_element_type=jnp.float32
    )
    sn = _rownorm(s)
    support = _proj(jnp.tanh(sn) * s / sn)
    pn = _rownorm(support)
    xt = jax.nn.relu(_artanh(pn) * support / pn)
    xtn = _rownorm(xt)
    out_ref[...] = _proj(jnp.tanh(xtn) * xt / xtn)


@jax.jit
def kernel(x, adj, W):
    hidden_e = pl.pallas_call(
        _prologue_kernel,
        out_shape=jax.ShapeDtypeStruct((N, D), jnp.float32),
    )(x, W)

    out = pl.pallas_call(
        _main_kernel,
        grid=(N // TM,),
        in_specs=[
            pl.BlockSpec((TM, N), lambda i: (i, 0)),
            pl.BlockSpec((N, D), lambda i: (0, 0)),
        ],
        out_specs=pl.BlockSpec((TM, D), lambda i: (i, 0)),
        out_shape=jax.ShapeDtypeStruct((N, D), jnp.float32),
        compiler_params=pltpu.CompilerParams(
            dimension_semantics=("arbitrary",),
        ),
    )(adj, hidden_e)
    return out


# speedup vs baseline: 1.0749x; 1.0749x over previous
import functools

import jax
import jax.numpy as jnp
from jax.experimental import pallas as pl
from jax.experimental.pallas import tpu as pltpu

N = 10000
D = 128
TM = 400

_EPS = 1e-5


def _artanh(x):
    x = jnp.clip(x, -1.0 + 1e-7, 1.0 - 1e-7)
    return 0.5 * (jnp.log1p(x) - jnp.log1p(-x))


def _rownorm(x):
    return jnp.maximum(
        jnp.sqrt(jnp.sum(x * x, axis=-1, keepdims=True)), 1e-15
    )


def _proj(x):
    n = _rownorm(x)
    maxn = 1.0 - _EPS
    return jnp.where(n > maxn, x / n * maxn, x)


def _prologue_kernel(x_ref, w_ref, he_ref):
    x = x_ref[...]
    w = w_ref[...]
    xn = _rownorm(x)
    mx = jax.lax.dot_general(
        x, w, (((1,), (1,)), ((), ())), preferred_element_type=jnp.float32
    )
    mxn = _rownorm(mx)
    hidden = jnp.tanh(mxn / xn * _artanh(xn)) * mx / mxn
    pn = _rownorm(hidden)
    he_ref[...] = _artanh(pn) * hidden / pn


def _main_kernel(adj_ref, he_ref, out_ref):
    s = jnp.dot(
        adj_ref[...], he_ref[...], preferred_element_type=jnp.float32
    )
    sn = _rownorm(s)
    support = _proj(jnp.tanh(sn) * s / sn)
    pn = _rownorm(support)
    xt = jax.nn.relu(_artanh(pn) * support / pn)
    xtn = _rownorm(xt)
    out_ref[...] = _proj(jnp.tanh(xtn) * xt / xtn)


@jax.jit
def kernel(x, adj, W):
    hidden_e = pl.pallas_call(
        _prologue_kernel,
        out_shape=jax.ShapeDtypeStruct((N, D), jnp.float32),
    )(x, W)

    out = pl.pallas_call(
        _main_kernel,
        grid=(N // TM,),
        in_specs=[
            pl.BlockSpec((TM, N), lambda i: (i, 0)),
            pl.BlockSpec((N, D), lambda i: (0, 0)),
        ],
        out_specs=pl.BlockSpec((TM, D), lambda i: (i, 0)),
        out_shape=jax.ShapeDtypeStruct((N, D), jnp.float32),
        compiler_params=pltpu.CompilerParams(
            dimension_semantics=("arbitrary",),
        ),
    )(adj, hidden_e)
    return out


# speedup vs baseline: 1.1078x; 1.0306x over previous
import jax
import jax.numpy as jnp
from jax.experimental import pallas as pl
from jax.experimental.pallas import tpu as pltpu

N = 10000
D = 128
TM = 400

_EPS = 1e-5


def _artanh(x):
    x = jnp.clip(x, -1.0 + 1e-7, 1.0 - 1e-7)
    return 0.5 * (jnp.log1p(x) - jnp.log1p(-x))


def _rownorm(x):
    return jnp.maximum(
        jnp.sqrt(jnp.sum(x * x, axis=-1, keepdims=True)), 1e-15
    )


def _proj(x):
    n = _rownorm(x)
    maxn = 1.0 - _EPS
    return jnp.where(n > maxn, x / n * maxn, x)


def _fused_kernel(x_ref, w_ref, adj_ref, out_ref, he_ref):
    @pl.when(pl.program_id(0) == 0)
    def _prologue():
        w = w_ref[...]
        chunk = 1000

        def body(c, _):
            x = x_ref[pl.ds(c * chunk, chunk), :]
            xn = _rownorm(x)
            mx = jax.lax.dot_general(
                x, w, (((1,), (1,)), ((), ())),
                preferred_element_type=jnp.float32,
            )
            mxn = _rownorm(mx)
            hidden = jnp.tanh(mxn / xn * _artanh(xn)) * mx / mxn
            pn = _rownorm(hidden)
            he_ref[pl.ds(c * chunk, chunk), :] = _artanh(pn) * hidden / pn
            return 0

        jax.lax.fori_loop(0, N // chunk, body, 0)

    s = jnp.dot(
        adj_ref[...], he_ref[...], preferred_element_type=jnp.float32
    )
    sn = _rownorm(s)
    support = _proj(jnp.tanh(sn) * s / sn)
    pn = _rownorm(support)
    xt = jax.nn.relu(_artanh(pn) * support / pn)
    xtn = _rownorm(xt)
    out_ref[...] = _proj(jnp.tanh(xtn) * xt / xtn)


@jax.jit
def kernel(x, adj, W):
    return pl.pallas_call(
        _fused_kernel,
        grid=(N // TM,),
        in_specs=[
            pl.BlockSpec((N, D), lambda i: (0, 0)),
            pl.BlockSpec((D, D), lambda i: (0, 0)),
            pl.BlockSpec((TM, N), lambda i: (i, 0)),
        ],
        out_specs=pl.BlockSpec((TM, D), lambda i: (i, 0)),
        out_shape=jax.ShapeDtypeStruct((N, D), jnp.float32),
        scratch_shapes=[pltpu.VMEM((N, D), jnp.float32)],
        compiler_params=pltpu.CompilerParams(
            dimension_semantics=("arbitrary",),
        ),
    )(x, W, adj)


# speedup vs baseline: 1.1130x; 1.0047x over previous
import jax
import jax.numpy as jnp
from jax.experimental import pallas as pl
from jax.experimental.pallas import tpu as pltpu

N = 10000
D = 128
TM = 400

import math

_EPS = 1e-5
_ATANH_MAXN = 0.5 * (math.log1p(1.0 - _EPS) - math.log1p(-(1.0 - _EPS)))


def _artanh(x):
    x = jnp.clip(x, -1.0 + 1e-7, 1.0 - 1e-7)
    return 0.5 * (jnp.log1p(x) - jnp.log1p(-x))


def _rownorm(x):
    return jnp.maximum(
        jnp.sqrt(jnp.sum(x * x, axis=-1, keepdims=True)), 1e-15
    )


def _proj(x):
    n = _rownorm(x)
    maxn = 1.0 - _EPS
    return jnp.where(n > maxn, x / n * maxn, x)


def _fused_kernel(x_ref, w_ref, adj_ref, out_ref, he_ref):
    @pl.when(pl.program_id(0) == 0)
    def _prologue():
        w = w_ref[...]
        chunk = 1000

        def body(c, _):
            x = x_ref[pl.ds(c * chunk, chunk), :]
            xn = _rownorm(x)
            mx = jax.lax.dot_general(
                x, w, (((1,), (1,)), ((), ())),
                preferred_element_type=jnp.float32,
            )
            mxn = _rownorm(mx)
            hidden = jnp.tanh(mxn / xn * _artanh(xn)) * mx / mxn
            pn = _rownorm(hidden)
            he_ref[pl.ds(c * chunk, chunk), :] = _artanh(pn) * hidden / pn
            return 0

        jax.lax.fori_loop(0, N // chunk, body, 0)

    s = jnp.dot(
        adj_ref[...], he_ref[...],
        preferred_element_type=jnp.float32,
        precision=jax.lax.Precision.DEFAULT,
    )
    sn = _rownorm(s)
    xt = jax.nn.relu((jnp.minimum(sn, _ATANH_MAXN) / sn) * s)
    xtn = _rownorm(xt)
    out_ref[...] = _proj(jnp.tanh(xtn) * xt / xtn)


@jax.jit
def kernel(x, adj, W):
    return pl.pallas_call(
        _fused_kernel,
        grid=(N // TM,),
        in_specs=[
            pl.BlockSpec((N, D), lambda i: (0, 0)),
            pl.BlockSpec((D, D), lambda i: (0, 0)),
            pl.BlockSpec((TM, N), lambda i: (i, 0)),
        ],
        out_specs=pl.BlockSpec((TM, D), lambda i: (i, 0)),
        out_shape=jax.ShapeDtypeStruct((N, D), jnp.float32),
        scratch_shapes=[pltpu.VMEM((N, D), jnp.float32)],
        compiler_params=pltpu.CompilerParams(
            dimension_semantics=("arbitrary",),
        ),
    )(x, W, adj)
